# baseline (speedup 1.0000x reference)
import functools

import jax
import jax.numpy as jnp
from jax import lax
from jax.experimental import pallas as pl
from jax.experimental.pallas import tpu as pltpu
from jax.experimental.pallas import tpu_sc as plsc

_B = 16384
_N = 1000
_L = 16

_info = plsc.get_sparse_core_info()
_NC = _info.num_cores
_NS = _info.num_subcores
_NW = _NC * _NS
_BPW = _B // _NW


def _sc_body(u_hbm, v_hbm, tab_hbm, out_hbm, u_v, idx_v, val_v, sem):
    wid = lax.axis_index("s") * _NC + lax.axis_index("c")
    base = wid * _BPW

    pltpu.sync_copy(u_hbm.at[pl.ds(base, _BPW)], u_v)
    pltpu.sync_copy(v_hbm.at[pl.ds(base, _BPW)], idx_v)

    def flat_idx(i, carry):
        s = pl.ds(i * _L, _L)
        idx_v[s] = u_v[s] * _N + idx_v[s]
        return carry

    lax.fori_loop(0, _BPW // _L, flat_idx, 0)

    pltpu.async_copy(tab_hbm.at[idx_v], val_v, sem).wait()

    def sigmoid_chunk(i, carry):
        s = pl.ds(i * _L, _L)
        x = val_v[s]
        val_v[s] = 1.0 / (1.0 + jnp.exp(-x))
        return carry

    lax.fori_loop(0, _BPW // _L, sigmoid_chunk, 0)

    pltpu.sync_copy(val_v, out_hbm.at[pl.ds(base, _BPW)])


_sc_call = functools.partial(
    pl.kernel,
    out_type=jax.ShapeDtypeStruct((_B,), jnp.float32),
    mesh=plsc.VectorSubcoreMesh(core_axis_name="c", subcore_axis_name="s"),
    scratch_types=[
        pltpu.VMEM((_BPW,), jnp.int32),
        pltpu.VMEM((_BPW,), jnp.int32),
        pltpu.VMEM((_BPW,), jnp.float32),
        pltpu.SemaphoreType.DMA,
    ],
)(_sc_body)


@jax.jit
def kernel(u, v, morphisms_logits):
    table = morphisms_logits.reshape(-1)
    return _sc_call(u.astype(jnp.int32), v.astype(jnp.int32), table)


# speedup vs baseline: 1.0349x; 1.0349x over previous
import functools

import jax
import jax.numpy as jnp
from jax import lax
from jax.experimental import pallas as pl
from jax.experimental.pallas import tpu as pltpu
from jax.experimental.pallas import tpu_sc as plsc

_B = 16384
_N = 1000
_L = 16

_info = plsc.get_sparse_core_info()
_NC = _info.num_cores
_NS = _info.num_subcores
_NW = _NC * _NS
_BPW = _B // _NW


_H = _BPW // 2


def _sc_body(u_hbm, v_hbm, tab_hbm, out_hbm, u_v, idx_v, val_v, sem_u, sem_v,
             sem_g0, sem_g1, sem_o):
    wid = lax.axis_index("s") * _NC + lax.axis_index("c")
    base = wid * _BPW

    cp_u = pltpu.async_copy(u_hbm.at[pl.ds(base, _BPW)], u_v, sem_u)
    cp_v = pltpu.async_copy(v_hbm.at[pl.ds(base, _BPW)], idx_v, sem_v)
    cp_u.wait()
    cp_v.wait()

    for i in range(_BPW // _L):
        s = pl.ds(i * _L, _L)
        idx_v[s] = u_v[s] * _N + idx_v[s]

    g0 = pltpu.async_copy(tab_hbm.at[idx_v.at[pl.ds(0, _H)]],
                          val_v.at[pl.ds(0, _H)], sem_g0)
    g1 = pltpu.async_copy(tab_hbm.at[idx_v.at[pl.ds(_H, _H)]],
                          val_v.at[pl.ds(_H, _H)], sem_g1)
    g0.wait()
    for i in range(_H // _L):
        s = pl.ds(i * _L, _L)
        x = val_v[s]
        val_v[s] = 1.0 / (1.0 + jnp.exp(-x))
    g1.wait()
    for i in range(_H // _L, _BPW // _L):
        s = pl.ds(i * _L, _L)
        x = val_v[s]
        val_v[s] = 1.0 / (1.0 + jnp.exp(-x))

    pltpu.async_copy(val_v, out_hbm.at[pl.ds(base, _BPW)], sem_o).wait()


_sc_call = functools.partial(
    pl.kernel,
    out_type=jax.ShapeDtypeStruct((_B,), jnp.float32),
    mesh=plsc.VectorSubcoreMesh(core_axis_name="c", subcore_axis_name="s"),
    scratch_types=[
        pltpu.VMEM((_BPW,), jnp.int32),
        pltpu.VMEM((_BPW,), jnp.int32),
        pltpu.VMEM((_BPW,), jnp.float32),
        pltpu.SemaphoreType.DMA,
        pltpu.SemaphoreType.DMA,
        pltpu.SemaphoreType.DMA,
        pltpu.SemaphoreType.DMA,
        pltpu.SemaphoreType.DMA,
    ],
)(_sc_body)


@jax.jit
def kernel(u, v, morphisms_logits):
    table = morphisms_logits.reshape(-1)
    return _sc_call(u.astype(jnp.int32), v.astype(jnp.int32), table)
